# baseline (speedup 1.0000x reference)
import functools

import jax
import jax.numpy as jnp
import numpy as np
from jax import lax
from jax.experimental import pallas as pl
from jax.experimental.pallas import tpu as pltpu
from jax.experimental.pallas import tpu_sc as plsc

H = 384
W = 384
C = 96
NB = 4
NPIX = NB * H * W
NPAIR = NPIX // 2
NC, NS, L = 2, 16, 16
NW = NC * NS
ROWS_PER_W = NPIX // NW
IMROWS_PER_W = ROWS_PER_W // W
K = 96
CHUNKS_PER_ROW = W // K
GROUPS = K // L
NSETS = 2

_SCALE = np.float32(382.0)
_STEP = np.float32(2.0 / 383.0)


def _floor_i32(v):
    c = v.astype(jnp.int32)
    return jnp.where(c.astype(jnp.float32) > v, c - 1, c)


def _bf16_round(v):
    bits = lax.bitcast_convert_type(v, jnp.int32)
    r = (bits + 0x7FFF + ((bits >> 16) & 1)) & np.int32(-65536)
    return lax.bitcast_convert_type(r, jnp.float32)


def _make_sc_call():
    mesh = plsc.VectorSubcoreMesh(
        core_axis_name="c", subcore_axis_name="s",
        num_cores=NC, num_subcores=NS)

    scratch = [pltpu.VMEM((NB, L), jnp.float32)]
    for _ in range(NSETS):
        scratch += [
            pltpu.VMEM((K,), jnp.int32),
            pltpu.VMEM((K,), jnp.int32),
            pltpu.VMEM((K,), jnp.float32),
            pltpu.VMEM((K,), jnp.float32),
            pltpu.VMEM((K,), jnp.float32),
            pltpu.VMEM((K,), jnp.float32),
            pltpu.VMEM((K, 2 * C), jnp.float32),
            pltpu.VMEM((K, 2 * C), jnp.float32),
            pltpu.VMEM((K, C), jnp.float32),
            pltpu.SemaphoreType.DMA,
            pltpu.SemaphoreType.DMA,
        ]

    @functools.partial(
        pl.kernel,
        out_type=jax.ShapeDtypeStruct((NPIX, C), jnp.float32),
        mesh=mesh,
        scratch_types=scratch,
        compiler_params=pltpu.CompilerParams(use_tc_tiling_on_sc=False),
    )
    def sc_sample(pairs_hbm, th_hbm, out_hbm, th_v, *bufs):
        sets = [bufs[i * 11:(i + 1) * 11] for i in range(NSETS)]
        wid = lax.axis_index("s") * NC + lax.axis_index("c")
        pltpu.sync_copy(th_hbm, th_v)
        base_row = wid * ROWS_PER_W
        b = wid >> 3
        bbase = b * (H * W)
        i0 = wid * IMROWS_PER_W - b * H

        tvec = th_v[b, :]

        def tsplat(k):
            return _bf16_round(jnp.full((L,), tvec[k], jnp.float32))
        t0, t1, t2, t3, t4, t5 = (tsplat(k) for k in range(6))
        iota = lax.iota(jnp.int32, 16)

        def imrow_body(ri, carry):
            i_ = i0 + ri
            yt = _bf16_round(
                jnp.full((L,), i_, jnp.int32).astype(jnp.float32) * _STEP - 1.0)
            ty_x = t1 * yt + t2
            ty_y = t4 * yt + t5
            outbase = base_row + ri * W

            gcp = [None] * NSETS
            ocp = [None] * NSETS

            def fire(ch):
                s = ch % NSETS
                (ia_v, ib_v, wa_v, wb_v, wc_v, wd_v,
                 ap, bp, ov, gsem, osem) = sets[s]
                if ocp[s] is not None:
                    ocp[s].wait()
                    ocp[s] = None
                for g in range(GROUPS):
                    sl = pl.ds(g * L, L)
                    j_ = iota + (ch * K + g * L)
                    xt = _bf16_round(j_.astype(jnp.float32) * _STEP - 1.0)
                    x = t0 * xt + ty_x
                    y = t3 * xt + ty_y
                    xs = 0.5 * ((x + 1.0) * _SCALE)
                    ys = 0.5 * ((y + 1.0) * _SCALE)
                    x0 = _floor_i32(xs)
                    y0 = _floor_i32(ys)
                    x0c = jnp.clip(x0, 0, W - 1)
                    x1c = jnp.clip(x0 + 1, 0, W - 1)
                    y0c = jnp.clip(y0, 0, H - 1)
                    y1c = jnp.clip(y0 + 1, 0, H - 1)
                    x0f = x0c.astype(jnp.float32)
                    x1f = x1c.astype(jnp.float32)
                    y0f = y0c.astype(jnp.float32)
                    y1f = y1c.astype(jnp.float32)
                    qa = bbase + y0c * W + x0c
                    qb = bbase + y1c * W + x0c
                    ia_v[sl] = (qa >> 1) + (qa & 1) * NPAIR
                    ib_v[sl] = (qb >> 1) + (qb & 1) * NPAIR
                    live = x1c > x0c
                    zero = jnp.full((L,), 0.0, jnp.float32)
                    wa_v[sl] = jnp.where(live, (x1f - xs) * (y1f - ys), zero)
                    wb_v[sl] = jnp.where(live, (x1f - xs) * (ys - y0f), zero)
                    wc_v[sl] = jnp.where(live, (xs - x0f) * (y1f - ys), zero)
                    wd_v[sl] = jnp.where(live, (xs - x0f) * (ys - y0f), zero)
                gcp[s] = (pltpu.async_copy(pairs_hbm.at[ia_v], ap, gsem),
                          pltpu.async_copy(pairs_hbm.at[ib_v], bp, gsem))

            fire(0)
            for ch in range(CHUNKS_PER_ROW):
                s = ch % NSETS
                (ia_v, ib_v, wa_v, wb_v, wc_v, wd_v,
                 ap, bp, ov, gsem, osem) = sets[s]
                for cp in gcp[s]:
                    cp.wait()
                if ch + 1 < CHUNKS_PER_ROW:
                    fire(ch + 1)

                def group_body(g, gcarry):
                    gb = g * L
                    wga = wa_v[pl.ds(gb, L)]
                    wgb = wb_v[pl.ds(gb, L)]
                    wgc = wc_v[pl.ds(gb, L)]
                    wgd = wd_v[pl.ds(gb, L)]
                    for lane in range(L):
                        r = gb + lane
                        wav = jnp.full((L,), wga[lane], jnp.float32)
                        wbv = jnp.full((L,), wgb[lane], jnp.float32)
                        wcv = jnp.full((L,), wgc[lane], jnp.float32)
                        wdv = jnp.full((L,), wgd[lane], jnp.float32)
                        for cc in range(C // L):
                            c0 = pl.ds(cc * L, L)
                            c1 = pl.ds(C + cc * L, L)
                            ov[r, c0] = ((wav * ap[r, c0] + wbv * bp[r, c0])
                                         + wcv * ap[r, c1]) + wdv * bp[r, c1]
                    return gcarry
                lax.fori_loop(0, GROUPS, group_body, 0)
                ocp[s] = pltpu.async_copy(
                    ov, out_hbm.at[pl.ds(outbase + ch * K, K)], osem)
            for s in range(NSETS):
                if ocp[s] is not None:
                    ocp[s].wait()
            return carry
        lax.fori_loop(0, IMROWS_PER_W, imrow_body, 0)

    return sc_sample


_SC_SAMPLE = _make_sc_call()


def kernel(X, theta):
    flat = X.reshape(-1)
    flatp = jnp.concatenate([flat, jnp.zeros((C,), jnp.float32)])
    pairs = jnp.concatenate([flatp[:-C].reshape(NPAIR, 2 * C),
                             flatp[C:].reshape(NPAIR, 2 * C)], axis=0)
    th = jnp.pad(theta.astype(jnp.float32), ((0, 0), (0, L - 6)))
    out = _SC_SAMPLE(pairs, th)
    return out.reshape(NB, H, W, C)
